# baseline (speedup 1.0000x reference)
import functools

import jax
import jax.numpy as jnp
from jax import lax
from jax.experimental import pallas as pl
from jax.experimental.pallas import tpu as pltpu
from jax.experimental.pallas import tpu_sc as plsc

VOCAB = 100000
EMBED_DIM = 64
CONTEXT = 50
HIDDEN = 128

BV = 2048
NB = (VOCAB + BV - 1) // BV
VPAD = NB * BV

_NEG = -1e30


@functools.cache
def _make_sc_gather():
    mesh = plsc.VectorSubcoreMesh(core_axis_name="c", subcore_axis_name="s")

    @functools.partial(
        pl.kernel,
        out_type=jax.ShapeDtypeStruct((CONTEXT, EMBED_DIM), jnp.float32),
        mesh=mesh,
        scratch_types=[
            pltpu.VMEM((CONTEXT,), jnp.int32),
            pltpu.VMEM((CONTEXT, EMBED_DIM), jnp.float32),
            pltpu.SemaphoreType.DMA,
        ],
        compiler_params=pltpu.CompilerParams(use_tc_tiling_on_sc=False),
    )
    def _sc_gather(idx_hbm, table_hbm, out_hbm, idx_v, rows_v, sem):
        wid = lax.axis_index("s") * 2 + lax.axis_index("c")

        @pl.when(wid == 0)
        def _():
            pltpu.sync_copy(idx_hbm, idx_v)
            pltpu.async_copy(table_hbm.at[idx_v], rows_v, sem).wait()
            pltpu.sync_copy(rows_v, out_hbm)

    return _sc_gather


def _mlp_body(e_ref, w1_ref, b1_ref, w2_ref, b2_ref, o_ref, h_ref, ms_ref):
    j = pl.program_id(0)

    @pl.when(j == 0)
    def _():
        h = jnp.dot(e_ref[...], w1_ref[...],
                    preferred_element_type=jnp.float32) + b1_ref[...]
        h_ref[...] = jnp.maximum(h, 0.0)
        ms_ref[0] = _NEG
        ms_ref[1] = 0.0

    logits = jnp.dot(h_ref[...], w2_ref[...],
                     preferred_element_type=jnp.float32) + b2_ref[...]
    col = j * BV + lax.broadcasted_iota(jnp.int32, (1, BV), 1)
    logits = jnp.where(col < VOCAB, logits, _NEG)
    o_ref[:, pl.ds(j * BV, BV)] = logits

    m_old = ms_ref[0]
    s_old = ms_ref[1]
    m_new = jnp.maximum(m_old, jnp.max(logits))
    s_new = s_old * jnp.exp(m_old - m_new) + jnp.sum(jnp.exp(logits - m_new))
    ms_ref[0] = m_new
    ms_ref[1] = s_new

    @pl.when(j == NB - 1)
    def _():
        o_ref[...] = o_ref[...] - (m_new + jnp.log(s_new))


_mlp_call = pl.pallas_call(
    _mlp_body,
    grid=(NB,),
    in_specs=[
        pl.BlockSpec((1, CONTEXT * EMBED_DIM), lambda j: (0, 0)),
        pl.BlockSpec((CONTEXT * EMBED_DIM, HIDDEN), lambda j: (0, 0)),
        pl.BlockSpec((1, HIDDEN), lambda j: (0, 0)),
        pl.BlockSpec((HIDDEN, BV), lambda j: (0, j)),
        pl.BlockSpec((1, BV), lambda j: (0, j)),
    ],
    out_specs=pl.BlockSpec((1, VPAD), lambda j: (0, 0)),
    out_shape=jax.ShapeDtypeStruct((1, VPAD), jnp.float32),
    scratch_shapes=[
        pltpu.VMEM((1, HIDDEN), jnp.float32),
        pltpu.SMEM((2,), jnp.float32),
    ],
)


def kernel(inputs, table, W1, b1, W2, b2):
    idx = inputs.astype(jnp.int32)
    embeds = _make_sc_gather()(idx, table).reshape(1, CONTEXT * EMBED_DIM)
    out = _mlp_call(embeds, W1, b1.reshape(1, HIDDEN), W2,
                    b2.reshape(1, VOCAB))
    return out[:, :VOCAB]


# speedup vs baseline: 1.4095x; 1.4095x over previous
import jax
import jax.numpy as jnp
from jax import lax
from jax.experimental import pallas as pl
from jax.experimental.pallas import tpu as pltpu

VOCAB = 100000
EMBED_DIM = 64
CONTEXT = 50
HIDDEN = 128

BV = 4096
NB = (VOCAB + BV - 1) // BV
VPAD = NB * BV

_NEG = -1e30


def _body(idx_ref, *refs):
    row_refs = refs[:CONTEXT]
    w1_ref, b1_ref, w2_ref, b2_ref, o_ref, h_ref, ms_ref = refs[CONTEXT:]
    j = pl.program_id(0)

    @pl.when(j == 0)
    def _():
        h = b1_ref[...]
        sub = lax.broadcasted_iota(jnp.int32, (8, 1), 0)
        for k in range(CONTEXT):
            slab = row_refs[k][...]
            row = jnp.sum(jnp.where(sub == idx_ref[k] % 8, slab, 0.0),
                          axis=0, keepdims=True)
            h = h + jnp.dot(row,
                            w1_ref[pl.ds(k * EMBED_DIM, EMBED_DIM), :],
                            preferred_element_type=jnp.float32)
        h_ref[...] = jnp.maximum(h, 0.0)
        ms_ref[0] = _NEG
        ms_ref[1] = 0.0

    logits = jnp.dot(h_ref[...], w2_ref[...],
                     preferred_element_type=jnp.float32) + b2_ref[...]
    col = j * BV + lax.broadcasted_iota(jnp.int32, (1, BV), 1)
    logits = jnp.where(col < VOCAB, logits, _NEG)
    o_ref[:, pl.ds(j * BV, BV)] = logits

    m_old = ms_ref[0]
    s_old = ms_ref[1]
    m_new = jnp.maximum(m_old, jnp.max(logits))
    s_new = s_old * jnp.exp(m_old - m_new) + jnp.sum(jnp.exp(logits - m_new))
    ms_ref[0] = m_new
    ms_ref[1] = s_new

    @pl.when(j == NB - 1)
    def _():
        o_ref[...] = o_ref[...] - (m_new + jnp.log(s_new))


def _row_spec(k):
    return pl.BlockSpec((8, EMBED_DIM), lambda j, idx, _k=k: (idx[_k] // 8, 0))


_grid_spec = pltpu.PrefetchScalarGridSpec(
    num_scalar_prefetch=1,
    grid=(NB,),
    in_specs=[
        *[_row_spec(k) for k in range(CONTEXT)],
        pl.BlockSpec((CONTEXT * EMBED_DIM, HIDDEN), lambda j, idx: (0, 0)),
        pl.BlockSpec((1, HIDDEN), lambda j, idx: (0, 0)),
        pl.BlockSpec((HIDDEN, BV), lambda j, idx: (0, j)),
        pl.BlockSpec((1, BV), lambda j, idx: (0, j)),
    ],
    out_specs=pl.BlockSpec((1, VPAD), lambda j, idx: (0, 0)),
    scratch_shapes=[
        pltpu.VMEM((1, HIDDEN), jnp.float32),
        pltpu.SMEM((2,), jnp.float32),
    ],
)

_mlp_call = pl.pallas_call(
    _body,
    grid_spec=_grid_spec,
    out_shape=jax.ShapeDtypeStruct((1, VPAD), jnp.float32),
)


def kernel(inputs, table, W1, b1, W2, b2):
    idx = inputs.astype(jnp.int32)
    out = _mlp_call(idx, *([table] * CONTEXT), W1, b1.reshape(1, HIDDEN),
                    W2, b2.reshape(1, VOCAB))
    return out[:, :VOCAB]


# speedup vs baseline: 1.5566x; 1.1043x over previous
import jax
import jax.numpy as jnp
from jax import lax
from jax.experimental import pallas as pl
from jax.experimental.pallas import tpu as pltpu

VOCAB = 100000
EMBED_DIM = 64
CONTEXT = 50
HIDDEN = 128

BV = 8192
NB = (VOCAB + BV - 1) // BV
VPAD = NB * BV

_NEG = -1e30


def _body(idx_ref, *refs):
    row_refs = refs[:CONTEXT]
    w1_ref, b1_ref, w2_ref, b2_ref, o_ref, h_ref, ms_ref = refs[CONTEXT:]
    j = pl.program_id(0)

    @pl.when(j == 0)
    def _():
        h = b1_ref[...]
        sub = lax.broadcasted_iota(jnp.int32, (8, 1), 0)
        for k in range(CONTEXT):
            slab = row_refs[k][...]
            row = jnp.sum(jnp.where(sub == idx_ref[k] % 8, slab, 0.0),
                          axis=0, keepdims=True)
            h = h + jnp.dot(row,
                            w1_ref[pl.ds(k * EMBED_DIM, EMBED_DIM), :],
                            preferred_element_type=jnp.float32)
        h_ref[...] = jnp.maximum(h, 0.0)
        ms_ref[0] = _NEG
        ms_ref[1] = 0.0

    logits = jnp.dot(h_ref[...], w2_ref[...],
                     preferred_element_type=jnp.float32) + b2_ref[...]
    col = j * BV + lax.broadcasted_iota(jnp.int32, (1, BV), 1)
    logits = jnp.where(col < VOCAB, logits, _NEG)
    o_ref[:, pl.ds(j * BV, BV)] = logits

    m_old = ms_ref[0]
    s_old = ms_ref[1]
    m_new = jnp.maximum(m_old, jnp.max(logits))
    s_new = s_old * jnp.exp(m_old - m_new) + jnp.sum(jnp.exp(logits - m_new))
    ms_ref[0] = m_new
    ms_ref[1] = s_new

    @pl.when(j == NB - 1)
    def _():
        o_ref[...] = o_ref[...] - (m_new + jnp.log(s_new))


def _row_spec(k):
    return pl.BlockSpec((8, EMBED_DIM), lambda j, idx, _k=k: (idx[_k] // 8, 0))


_grid_spec = pltpu.PrefetchScalarGridSpec(
    num_scalar_prefetch=1,
    grid=(NB,),
    in_specs=[
        *[_row_spec(k) for k in range(CONTEXT)],
        pl.BlockSpec((CONTEXT * EMBED_DIM, HIDDEN), lambda j, idx: (0, 0)),
        pl.BlockSpec((1, HIDDEN), lambda j, idx: (0, 0)),
        pl.BlockSpec((HIDDEN, BV), lambda j, idx: (0, j)),
        pl.BlockSpec((1, BV), lambda j, idx: (0, j)),
    ],
    out_specs=pl.BlockSpec((1, VPAD), lambda j, idx: (0, 0)),
    scratch_shapes=[
        pltpu.VMEM((1, HIDDEN), jnp.float32),
        pltpu.SMEM((2,), jnp.float32),
    ],
)

_mlp_call = pl.pallas_call(
    _body,
    grid_spec=_grid_spec,
    out_shape=jax.ShapeDtypeStruct((1, VPAD), jnp.float32),
)


def kernel(inputs, table, W1, b1, W2, b2):
    idx = inputs.astype(jnp.int32)
    out = _mlp_call(idx, *([table] * CONTEXT), W1, b1.reshape(1, HIDDEN),
                    W2, b2.reshape(1, VOCAB))
    return out[:, :VOCAB]
